# baseline (speedup 1.0000x reference)
import functools

import jax
import jax.numpy as jnp
from jax import lax
from jax.experimental import pallas as pl
from jax.experimental.pallas import tpu as pltpu
from jax.experimental.pallas import tpu_sc as plsc

B = 16384
D = 64
NW = 32
BPW = B // NW


def _sc_kernel(y_hbm, x_hbm, idx_hbm, t_hbm, table_hbm, out_hbm,
               idx_v, rows_v, y_v, x_v, t_v, gsem):
    wid = lax.axis_index("s") * 2 + lax.axis_index("c")
    base = wid * BPW

    pltpu.sync_copy(idx_hbm.at[pl.ds(base, BPW)], idx_v)
    gather = pltpu.async_copy(table_hbm.at[idx_v], rows_v, gsem)

    pltpu.sync_copy(y_hbm.at[pl.ds(base, BPW)], y_v)
    pltpu.sync_copy(y_v, out_hbm.at[pl.ds(base, BPW), pl.ds(0, 8)])
    pltpu.sync_copy(x_hbm.at[pl.ds(base, BPW)], x_v)
    pltpu.sync_copy(x_v, out_hbm.at[pl.ds(base, BPW), pl.ds(8, 8)])
    pltpu.sync_copy(t_hbm.at[pl.ds(base, BPW)], t_v)
    pltpu.sync_copy(t_v, out_hbm.at[pl.ds(base, BPW), pl.ds(80, 4)])

    gather.wait()
    pltpu.sync_copy(rows_v, out_hbm.at[pl.ds(base, BPW), pl.ds(16, D)])


@jax.jit
def _run(y2, x2, idx, t, table):
    mesh = plsc.VectorSubcoreMesh(core_axis_name="c", subcore_axis_name="s")
    f = functools.partial(
        pl.kernel, mesh=mesh,
        compiler_params=pltpu.CompilerParams(use_tc_tiling_on_sc=False),
        out_type=jax.ShapeDtypeStruct((B, 84), jnp.float32),
        scratch_types=[
            pltpu.VMEM((BPW,), jnp.int32),
            pltpu.VMEM((BPW, D), jnp.float32),
            pltpu.VMEM((BPW, 8), jnp.float32),
            pltpu.VMEM((BPW, 8), jnp.float32),
            pltpu.VMEM((BPW, 4), jnp.float32),
            pltpu.SemaphoreType.DMA,
        ],
    )(_sc_kernel)
    return f(y2, x2, idx, t, table)


def kernel(gsp_y_osgb_fourier, gsp_x_osgb_fourier, gsp_id,
           gsp_5_min_time_utc_fourier, emb_table):
    y2 = gsp_y_osgb_fourier[:, 0]
    x2 = gsp_x_osgb_fourier[:, 0]
    idx = gsp_id.astype(jnp.int32)
    out = _run(y2, x2, idx, gsp_5_min_time_utc_fourier, emb_table)
    return out[:, None, :]
